# Initial kernel scaffold; baseline (speedup 1.0000x reference)
import jax
import jax.numpy as jnp
from jax.experimental import pallas as pl


def kernel(pred_rgI, eq_mat, ineq_mat, random_flip):
    raise NotImplementedError("write your pallas kernel here")


# speedup vs baseline: 1.0790x; 1.0790x over previous
import jax
import jax.numpy as jnp
from jax import lax
from jax.experimental import pallas as pl
from jax.experimental.pallas import tpu as pltpu
from jax.experimental.pallas import tpu_sc as plsc

W_LOSS = 1.0
W_INEQ = 1.0
MARGIN_EQ = 0.1
MARGIN_INEQ = 0.25

NC = 2
NS = 16
L = 16

ROWS = 512
COLS = 512
NJ = 20000
HALF = NJ // 2
CHUNK = 128
NROW = 80
PAD = NROW * CHUNK
GROUP = 8
NGROUP = NROW // GROUP


def _body(pred_hbm, eq_hbm, ineq_hbm, flip_hbm, out_hbm,
          mat_v, idx1_v, idx2_v, vals1_v, vals2_v, w_v, flip_v, out_v, sem):
    c = lax.axis_index("c")
    s = lax.axis_index("s")
    wid = s * NC + c
    img = s // 2
    is_ineq = s % 2

    pix_base = (img * 3 + 2) * (ROWS * COLS)

    pltpu.sync_copy(flip_hbm, flip_v)

    mat_off = pl.multiple_of(img * (NJ * 5) + c * (HALF * 5), 8)

    @pl.when(is_ineq == 0)
    def _():
        pltpu.sync_copy(eq_hbm.at[pl.ds(mat_off, HALF * 5)], mat_v)

    @pl.when(is_ineq == 1)
    def _():
        pltpu.sync_copy(ineq_hbm.at[pl.ds(mat_off, HALF * 5)], mat_v)

    lanes = jnp.arange(L, dtype=jnp.int32)
    zi = jnp.zeros((L,), jnp.int32)
    zf = jnp.zeros((L,), jnp.float32)
    flipb = plsc.load_gather(flip_v, [zi + img]) != 0

    def zero_pad(k, _):
        off = pl.multiple_of(HALF + k * L, 8)
        idx1_v[pl.ds(off, L)] = zi
        idx2_v[pl.ds(off, L)] = zi
        w_v[pl.ds(off, L)] = zf
        return 0

    lax.fori_loop(0, (PAD - HALF) // L, zero_pad, 0)

    col_idx = lanes * 5

    def mk_idx(yf, xf):
        y = (yf * float(ROWS)).astype(jnp.int32)
        y = jnp.minimum(jnp.maximum(y, 0), ROWS - 1)
        x = (xf * float(COLS)).astype(jnp.int32)
        x = jnp.minimum(jnp.maximum(x, 0), COLS - 1)
        x = jnp.where(flipb, (COLS - 1) - x, x)
        return pix_base + y * COLS + x

    def idx_body(i, _):
        base = i * (L * 5)
        ci = col_idx + base
        y1f = plsc.load_gather(mat_v, [ci])
        x1f = plsc.load_gather(mat_v, [ci + 1])
        y2f = plsc.load_gather(mat_v, [ci + 2])
        x2f = plsc.load_gather(mat_v, [ci + 3])
        w = plsc.load_gather(mat_v, [ci + 4])
        off = pl.multiple_of(i * L, 8)
        idx1_v[pl.ds(off, L)] = mk_idx(y1f, x1f)
        idx2_v[pl.ds(off, L)] = mk_idx(y2f, x2f)
        w_v[pl.ds(off, L)] = w
        return 0

    lax.fori_loop(0, HALF // L, idx_body, 0)

    def gather_group(g, _):
        cps = []
        for r in range(GROUP):
            off = pl.multiple_of((g * GROUP + r) * CHUNK, 8)
            cps.append(pltpu.async_copy(
                pred_hbm.at[idx1_v.at[pl.ds(off, CHUNK)]],
                vals1_v.at[pl.ds(off, CHUNK)], sem))
            cps.append(pltpu.async_copy(
                pred_hbm.at[idx2_v.at[pl.ds(off, CHUNK)]],
                vals2_v.at[pl.ds(off, CHUNK)], sem))
        for cp in cps:
            cp.wait()
        return 0

    lax.fori_loop(0, NGROUP, gather_group, 0)

    iv = (zi + is_ineq) != 0

    def acc_body(i, carry):
        acc, accw = carry
        off = pl.multiple_of(i * L, 8)
        v1 = vals1_v[pl.ds(off, L)]
        v2 = vals2_v[pl.ds(off, L)]
        w = w_v[pl.ds(off, L)]
        d = v1 - v2
        eq_t = jnp.maximum(jnp.abs(d) - MARGIN_EQ, 0.0)
        ineq_t = jnp.maximum(MARGIN_INEQ - d, 0.0)
        t = jnp.where(iv, ineq_t, eq_t) * w
        return acc + t, accw + w

    acc, accw = lax.fori_loop(0, PAD // L, acc_body, (zf, zf))
    num = jnp.sum(acc)
    den = jnp.sum(accw)

    out_v[...] = jnp.where(lanes == 0, num, jnp.where(lanes == 1, den, 0.0))
    pltpu.sync_copy(out_v, out_hbm.at[wid])


_SCRATCH = [
    pltpu.VMEM((HALF * 5,), jnp.float32),
    pltpu.VMEM((PAD,), jnp.int32),
    pltpu.VMEM((PAD,), jnp.int32),
    pltpu.VMEM((PAD,), jnp.float32),
    pltpu.VMEM((PAD,), jnp.float32),
    pltpu.VMEM((PAD,), jnp.float32),
    pltpu.VMEM((L,), jnp.int32),
    pltpu.VMEM((L,), jnp.float32),
    pltpu.SemaphoreType.DMA,
]


def _partials(pred_flat, eq_flat, ineq_flat, flip16):
    mesh = plsc.VectorSubcoreMesh(
        core_axis_name="c", subcore_axis_name="s", num_cores=NC,
        num_subcores=NS)
    f = pl.kernel(
        _body,
        out_type=jax.ShapeDtypeStruct((NC * NS, L), jnp.float32),
        mesh=mesh,
        scratch_types=_SCRATCH,
        compiler_params=pltpu.CompilerParams(needs_layout_passes=False),
    )
    return f(pred_flat, eq_flat, ineq_flat, flip16)


@jax.jit
def kernel(pred_rgI, eq_mat, ineq_mat, random_flip):
    n_img = pred_rgI.shape[0]
    pred_flat = pred_rgI.reshape(-1)
    eq_flat = eq_mat.reshape(-1)
    ineq_flat = ineq_mat.reshape(-1)
    flip16 = jnp.pad(random_flip.astype(jnp.int32), (0, 16 - n_img))

    parts = _partials(pred_flat, eq_flat, ineq_flat, flip16)

    num = parts[:, 0].reshape(n_img, 2, 2).sum(-1)
    den = parts[:, 1].reshape(n_img, 2, 2).sum(-1)
    per = num / jnp.maximum(den, 1e-6)
    per_img = (per[:, 0] + W_INEQ * per[:, 1]) / (1.0 + W_INEQ)
    return per_img.mean() * W_LOSS
